# Initial kernel scaffold; baseline (speedup 1.0000x reference)
import jax
import jax.numpy as jnp
from jax.experimental import pallas as pl


def kernel(node_feats, edge_index, etype, graph_ids, Wrel1, Wself1, b1, Wrel2, Wself2, b2, aw_W, aw_b, fc1_W, fc1_b, bn1_g, bn1_b, fc2_W, fc2_b, bn2_g, bn2_b, fc3_W, fc3_b, bn3_g, bn3_b, outA_W, outA_b, outB_W, outB_b):
    raise NotImplementedError("write your pallas kernel here")


# speedup vs baseline: 29.9796x; 29.9796x over previous
import functools

import jax
import jax.numpy as jnp
from jax import lax
from jax.experimental import pallas as pl
from jax.experimental.pallas import tpu as pltpu
from jax.experimental.pallas import tpu_sc as plsc

_N = 10000
_E = 320000
_D = 128
_R = 4
_T = 3
_G = 256
_H = 128

_NC = 2
_NS = 16
_NW = _NC * _NS
_EPW = _E // _NW
_C = 80
_NCH = _EPW // _C
_P = 5
_CPP = _NCH // _P
_NBUF = 2
_NAGG = 10112
_RPS = _NAGG // _NS
_ZR = 8

@functools.cache
def _make_edge_aggregate():
    mesh = plsc.VectorSubcoreMesh(
        core_axis_name="c", subcore_axis_name="s",
        num_cores=_NC, num_subcores=_NS)
    return functools.partial(
        pl.kernel,
        out_type=jax.ShapeDtypeStruct((_NC, _NAGG, _D), jnp.float32),
        mesh=mesh,
        scratch_types=[
            pltpu.VMEM((_CPP, _C), jnp.int32),
            pltpu.VMEM((_CPP, _C), jnp.int32),
            pltpu.VMEM((_CPP, _C), jnp.int32),
            pltpu.VMEM((_NBUF, _C, _D), jnp.float32),
            pltpu.VMEM((_ZR, _D), jnp.float32),
            pltpu.VMEM_SHARED((_NAGG, _D), jnp.float32),
            [pltpu.SemaphoreType.DMA] * _NBUF,
        ],
    )(_edge_aggregate_body)


def _edge_aggregate_body(tab, src_h, et_h, dst_h, out_h,
                         gib, etb, dstb, rbuf, zbuf, agg, sems):
    c = lax.axis_index("c")
    s = lax.axis_index("s")
    wid = s * _NC + c

    zvec = jnp.zeros((16,), jnp.float32)

    def _zfill(i, carry):
        zbuf[i // 8, pl.ds((i % 8) * 16, 16)] = zvec
        return carry
    lax.fori_loop(0, _ZR * (_D // 16), _zfill, 0)

    def _zcp(j, carry):
        pltpu.sync_copy(zbuf, agg.at[pl.ds(s * _RPS + j * _ZR, _ZR), :])
        return carry
    lax.fori_loop(0, _RPS // _ZR, _zcp, 0)
    plsc.subcore_barrier()

    def _fire(j, b):
        pltpu.async_copy(tab.at[gib.at[j]], rbuf.at[b], sems[b])

    def _wait(b):
        pltpu.make_async_copy(tab.at[pl.ds(0, _C)], rbuf.at[b], sems[b]).wait()

    def _scat(j, b):
        pltpu.sync_copy(rbuf.at[b], agg.at[dstb.at[j]], add=True)

    def _pass(p, carry):
        pltpu.sync_copy(src_h.at[wid, p], gib)
        pltpu.sync_copy(et_h.at[wid, p], etb)
        pltpu.sync_copy(dst_h.at[wid, p], dstb)

        def _gidx(i, carry2):
            j = i // (_C // 16)
            k = i % (_C // 16)
            sv = gib[j, pl.ds(k * 16, 16)]
            ev = etb[j, pl.ds(k * 16, 16)]
            gib[j, pl.ds(k * 16, 16)] = sv * (_R + 1) + ev
            return carry2
        lax.fori_loop(0, _CPP * (_C // 16), _gidx, 0)

        for b in range(_NBUF):
            _fire(b, b)

        def _outer(o, carry2):
            for b in range(_NBUF):
                j = o * _NBUF + b
                _wait(b)
                _scat(j, b)
                jn = j + _NBUF

                @pl.when(jn < _CPP)
                def _():
                    _fire(jn, b)
            return carry2
        lax.fori_loop(0, _CPP // _NBUF, _outer, 0)
        for j in range((_CPP // _NBUF) * _NBUF, _CPP):
            _wait(j % _NBUF)
            _scat(j, j % _NBUF)
        return carry
    lax.fori_loop(0, _P, _pass, 0)

    plsc.subcore_barrier()
    pltpu.sync_copy(agg.at[pl.ds(s * _RPS, _RPS), :],
                    out_h.at[c, pl.ds(s * _RPS, _RPS), :])


_BN = 1000


def _mm_body(x_ref, w_ref, o_ref):
    o_ref[...] = jnp.dot(x_ref[...], w_ref[...],
                         preferred_element_type=jnp.float32)


def _dense_matmul(x, w):
    n, k = x.shape
    m = w.shape[1]
    return pl.pallas_call(
        _mm_body,
        grid=(n // _BN,),
        in_specs=[pl.BlockSpec((_BN, k), lambda i: (i, 0)),
                  pl.BlockSpec((k, m), lambda i: (0, 0))],
        out_specs=pl.BlockSpec((_BN, m), lambda i: (i, 0)),
        out_shape=jax.ShapeDtypeStruct((n, m), jnp.float32),
    )(x, w)


def _layer_body(a0_ref, a1_ref, sp_ref, b_ref, w_ref, o_ref):
    h = jnp.maximum(a0_ref[...] + a1_ref[...] + sp_ref[...] + b_ref[...], 0.0)
    o_ref[...] = jnp.dot(h, w_ref[...], preferred_element_type=jnp.float32)


def _layer_matmul(a0, a1, sp, bias, w):
    n = a0.shape[0]
    m = w.shape[1]
    return pl.pallas_call(
        _layer_body,
        grid=(n // _BN,),
        in_specs=[pl.BlockSpec((_BN, _D), lambda i: (i, 0)),
                  pl.BlockSpec((_BN, _D), lambda i: (i, 0)),
                  pl.BlockSpec((_BN, _D), lambda i: (i, 0)),
                  pl.BlockSpec((1, _D), lambda i: (0, 0)),
                  pl.BlockSpec((_D, m), lambda i: (0, 0))],
        out_specs=pl.BlockSpec((_BN, m), lambda i: (i, 0)),
        out_shape=jax.ShapeDtypeStruct((n, m), jnp.float32),
    )(a0, a1, sp, bias, w)


def _finalh_body(a0_ref, a1_ref, sp_ref, b_ref, aw_ref, ab_ref, h_ref, w_ref):
    h = jnp.maximum(a0_ref[...] + a1_ref[...] + sp_ref[...] + b_ref[...], 0.0)
    h_ref[...] = h
    logits = jnp.dot(h, aw_ref[...], preferred_element_type=jnp.float32)
    w_ref[...] = jax.nn.sigmoid(logits + ab_ref[...])


def _final_h(a0, a1, sp, bias, awp, awb):
    n = a0.shape[0]
    return pl.pallas_call(
        _finalh_body,
        grid=(n // _BN,),
        in_specs=[pl.BlockSpec((_BN, _D), lambda i: (i, 0)),
                  pl.BlockSpec((_BN, _D), lambda i: (i, 0)),
                  pl.BlockSpec((_BN, _D), lambda i: (i, 0)),
                  pl.BlockSpec((1, _D), lambda i: (0, 0)),
                  pl.BlockSpec((_D, 128), lambda i: (0, 0)),
                  pl.BlockSpec((1, 128), lambda i: (0, 0))],
        out_specs=[pl.BlockSpec((_BN, _D), lambda i: (i, 0)),
                   pl.BlockSpec((_BN, 128), lambda i: (i, 0))],
        out_shape=[jax.ShapeDtypeStruct((n, _D), jnp.float32),
                   jax.ShapeDtypeStruct((n, 128), jnp.float32)],
    )(a0, a1, sp, bias, awp, awb)


_RBN = 1000
_RNB = _N // _RBN


def _readout_body(gid_ref, wm_ref, h_ref, o_ref):
    @pl.when(pl.program_id(0) == 0)
    def _():
        o_ref[...] = jnp.zeros_like(o_ref)

    gid = gid_ref[0]
    grange = lax.broadcasted_iota(jnp.int32, (_G, _RBN), 0)
    mask = (grange == gid).astype(jnp.float32)
    h = h_ref[...]
    upd = []
    for t in range(_T):
        hw = h * wm_ref[:, t:t + 1]
        upd.append(jnp.dot(mask, hw, preferred_element_type=jnp.float32))
    o_ref[...] += jnp.stack(upd, axis=0)


def _readout(gid3, wm, h):
    return pl.pallas_call(
        _readout_body,
        grid=(_RNB,),
        in_specs=[pl.BlockSpec((1, 1, _RBN), lambda i: (i, 0, 0)),
                  pl.BlockSpec((_RBN, 128), lambda i: (i, 0)),
                  pl.BlockSpec((_RBN, _D), lambda i: (i, 0))],
        out_specs=pl.BlockSpec((_T, _G, _D), lambda i: (0, 0, 0)),
        out_shape=jax.ShapeDtypeStruct((_T, _G, _D), jnp.float32),
    )(gid3, wm, h)


def _ffn_body(gf_ref, w1_ref, b1_ref, g1_ref, gb1_ref,
              w2_ref, b2_ref, g2_ref, gb2_ref,
              w3_ref, b3_ref, g3_ref, gb3_ref,
              wa_ref, ba_ref, wbp_ref, bbp_ref, o_ref):
    acc = jnp.zeros((_G, 128), jnp.float32)
    for t in range(_T):
        x = gf_ref[t]
        x = jnp.maximum(
            jnp.dot(x, w1_ref[t], preferred_element_type=jnp.float32)
            + b1_ref[t:t + 1], 0.0)
        x = x * g1_ref[t:t + 1] + gb1_ref[t:t + 1]
        x = jnp.maximum(
            jnp.dot(x, w2_ref[t], preferred_element_type=jnp.float32)
            + b2_ref[t:t + 1], 0.0)
        x = x * g2_ref[t:t + 1] + gb2_ref[t:t + 1]
        x = jnp.maximum(
            jnp.dot(x, w3_ref[t], preferred_element_type=jnp.float32)
            + b3_ref[t:t + 1], 0.0)
        x = x * g3_ref[t:t + 1] + gb3_ref[t:t + 1]
        hid = jnp.maximum(
            jnp.dot(x, wa_ref[t], preferred_element_type=jnp.float32)
            + ba_ref[t:t + 1], 0.0)
        acc = acc + jnp.dot(hid, wbp_ref[t],
                            preferred_element_type=jnp.float32)
    o_ref[...] = acc + bbp_ref[...]


def _ffn(gf, *ws):
    full = lambda a: pl.BlockSpec(a.shape, lambda: tuple(0 for _ in a.shape))
    return pl.pallas_call(
        _ffn_body,
        in_specs=[full(gf)] + [full(w) for w in ws],
        out_specs=pl.BlockSpec((_G, 128), lambda: (0, 0)),
        out_shape=jax.ShapeDtypeStruct((_G, 128), jnp.float32),
    )(gf, *ws)


def kernel(node_feats, edge_index, etype, graph_ids,
           Wrel1, Wself1, b1, Wrel2, Wself2, b2,
           aw_W, aw_b,
           fc1_W, fc1_b, bn1_g, bn1_b,
           fc2_W, fc2_b, bn2_g, bn2_b,
           fc3_W, fc3_b, bn3_g, bn3_b,
           outA_W, outA_b, outB_W, outB_b):
    f32 = jnp.float32
    src3 = edge_index[0].reshape(_NW, _P, _CPP, _C)
    dst3 = edge_index[1].reshape(_NW, _P, _CPP, _C)
    et3 = etype.reshape(_NW, _P, _CPP, _C)

    Wfull1 = jnp.concatenate(
        [Wrel1.transpose(1, 0, 2).reshape(_D, _R * _D), Wself1], axis=1)
    Wfull2 = jnp.concatenate(
        [Wrel2.transpose(1, 0, 2).reshape(_D, _R * _D), Wself2], axis=1)

    edge_agg = _make_edge_aggregate()
    unpad = lambda a: a[:_N]

    trans1 = _dense_matmul(node_feats, Wfull1)
    aggp1 = edge_agg(trans1.reshape(_N * (_R + 1), _D),
                     src3, et3, dst3)
    sp1 = trans1.reshape(_N, _R + 1, _D)[:, _R, :]

    trans2 = _layer_matmul(unpad(aggp1[0]), unpad(aggp1[1]), sp1,
                           b1.reshape(1, _D), Wfull2)
    aggp2 = edge_agg(trans2.reshape(_N * (_R + 1), _D),
                     src3, et3, dst3)
    sp2 = trans2.reshape(_N, _R + 1, _D)[:, _R, :]

    awp = jnp.zeros((_D, 128), f32).at[:, :_T].set(aw_W.T)
    awb = jnp.zeros((1, 128), f32).at[0, :_T].set(aw_b)
    h2, wm = _final_h(unpad(aggp2[0]), unpad(aggp2[1]), sp2,
                      b2.reshape(1, _D), awp, awb)

    gid3 = graph_ids.reshape(_RNB, 1, _RBN)
    gf = _readout(gid3, wm, h2)

    s0 = (1.0 + 1e-5) ** -0.5
    wbp = outB_W[:, :, None] * jnp.eye(_T, 128, dtype=f32)[:, None, :]
    bbp = jnp.zeros((1, 128), f32).at[0, :_T].set(outB_b)
    pred = _ffn(gf, fc1_W, fc1_b, bn1_g * s0, bn1_b,
                fc2_W, fc2_b, bn2_g * s0, bn2_b,
                fc3_W, fc3_b, bn3_g * s0, bn3_b,
                outA_W, outA_b, wbp, bbp)
    return pred[:, :_T]


# speedup vs baseline: 31.7007x; 1.0574x over previous
import functools

import jax
import jax.numpy as jnp
from jax import lax
from jax.experimental import pallas as pl
from jax.experimental.pallas import tpu as pltpu
from jax.experimental.pallas import tpu_sc as plsc

_N = 10000
_E = 320000
_D = 128
_R = 4
_T = 3
_G = 256
_H = 128

_NC = 2
_NS = 16
_NW = _NC * _NS
_EPW = _E // _NW
_C = 80
_NCH = _EPW // _C
_P = 5
_CPP = _NCH // _P
_NBUF = 2
_NAGG = 10112
_RPS = _NAGG // _NS
_ZR = 8

@functools.cache
def _make_edge_aggregate():
    mesh = plsc.VectorSubcoreMesh(
        core_axis_name="c", subcore_axis_name="s",
        num_cores=_NC, num_subcores=_NS)
    return functools.partial(
        pl.kernel,
        out_type=jax.ShapeDtypeStruct((_NC, _NAGG, _D), jnp.float32),
        mesh=mesh,
        scratch_types=[
            pltpu.VMEM((_CPP, _C), jnp.int32),
            pltpu.VMEM((_CPP, _C), jnp.int32),
            pltpu.VMEM((_NBUF, _C, _D), jnp.float32),
            pltpu.VMEM((_ZR, _D), jnp.float32),
            pltpu.VMEM_SHARED((_NAGG, _D), jnp.float32),
            [pltpu.SemaphoreType.DMA] * _NBUF,
        ],
    )(_edge_aggregate_body)


def _edge_aggregate_body(tab, gidx_h, dst_h, out_h,
                         gib, dstb, rbuf, zbuf, agg, sems):
    c = lax.axis_index("c")
    s = lax.axis_index("s")
    wid = s * _NC + c

    zvec = jnp.zeros((16,), jnp.float32)

    def _zfill(i, carry):
        zbuf[i // 8, pl.ds((i % 8) * 16, 16)] = zvec
        return carry
    lax.fori_loop(0, _ZR * (_D // 16), _zfill, 0)

    def _zcp(j, carry):
        pltpu.sync_copy(zbuf, agg.at[pl.ds(s * _RPS + j * _ZR, _ZR), :])
        return carry
    lax.fori_loop(0, _RPS // _ZR, _zcp, 0)
    plsc.subcore_barrier()

    def _fire(j, b):
        pltpu.async_copy(tab.at[gib.at[j]], rbuf.at[b], sems[b])

    def _wait(b):
        pltpu.make_async_copy(tab.at[pl.ds(0, _C)], rbuf.at[b], sems[b]).wait()

    def _scat(j, b):
        pltpu.sync_copy(rbuf.at[b], agg.at[dstb.at[j]], add=True)

    def _pass(p, carry):
        pltpu.sync_copy(gidx_h.at[wid, p], gib)
        pltpu.sync_copy(dst_h.at[wid, p], dstb)

        for b in range(_NBUF):
            _fire(b, b)

        def _outer(o, carry2):
            for b in range(_NBUF):
                j = o * _NBUF + b
                _wait(b)
                _scat(j, b)
                jn = j + _NBUF

                @pl.when(jn < _CPP)
                def _():
                    _fire(jn, b)
            return carry2
        lax.fori_loop(0, _CPP // _NBUF, _outer, 0)
        for j in range((_CPP // _NBUF) * _NBUF, _CPP):
            _wait(j % _NBUF)
            _scat(j, j % _NBUF)
        return carry
    lax.fori_loop(0, _P, _pass, 0)

    plsc.subcore_barrier()
    pltpu.sync_copy(agg.at[pl.ds(s * _RPS, _RPS), :],
                    out_h.at[c, pl.ds(s * _RPS, _RPS), :])


_BN = 1000


def _mm_body(x_ref, w_ref, o_ref):
    o_ref[...] = jnp.dot(x_ref[...], w_ref[...],
                         preferred_element_type=jnp.float32)


def _dense_matmul(x, w):
    n, k = x.shape
    m = w.shape[1]
    return pl.pallas_call(
        _mm_body,
        grid=(n // _BN,),
        in_specs=[pl.BlockSpec((_BN, k), lambda i: (i, 0)),
                  pl.BlockSpec((k, m), lambda i: (0, 0))],
        out_specs=pl.BlockSpec((_BN, m), lambda i: (i, 0)),
        out_shape=jax.ShapeDtypeStruct((n, m), jnp.float32),
    )(x, w)


def _gidx_body(s_ref, e_ref, o_ref):
    o_ref[...] = s_ref[...] * (_R + 1) + e_ref[...]


def _gather_idx(src2, et2):
    full = pl.BlockSpec((_E // 128, 128), lambda: (0, 0))
    return pl.pallas_call(
        _gidx_body,
        in_specs=[full, full],
        out_specs=full,
        out_shape=jax.ShapeDtypeStruct((_E // 128, 128), jnp.int32),
    )(src2, et2)


def _layer_body(a0_ref, a1_ref, tp_ref, b_ref, w_ref, o_ref):
    h = jnp.maximum(a0_ref[0] + a1_ref[0] + tp_ref[...] + b_ref[...], 0.0)
    o_ref[...] = jnp.dot(h, w_ref[...], preferred_element_type=jnp.float32)


def _layer_matmul(aggp, trans_prev, bias, w):
    m = w.shape[1]
    return pl.pallas_call(
        _layer_body,
        grid=(_N // _BN,),
        in_specs=[pl.BlockSpec((1, _BN, _D), lambda i: (0, i, 0)),
                  pl.BlockSpec((1, _BN, _D), lambda i: (1, i, 0)),
                  pl.BlockSpec((_BN, _D), lambda i: (i, _R)),
                  pl.BlockSpec((1, _D), lambda i: (0, 0)),
                  pl.BlockSpec((_D, m), lambda i: (0, 0))],
        out_specs=pl.BlockSpec((_BN, m), lambda i: (i, 0)),
        out_shape=jax.ShapeDtypeStruct((_N, m), jnp.float32),
    )(aggp, aggp, trans_prev, bias, w)


_RBN = 1000
_RNB = _N // _RBN


def _freadout_body(a0_ref, a1_ref, tp_ref, b_ref, aw_ref, ab_ref, gid_ref,
                   o_ref):
    @pl.when(pl.program_id(0) == 0)
    def _():
        o_ref[...] = jnp.zeros_like(o_ref)

    h = jnp.maximum(a0_ref[0] + a1_ref[0] + tp_ref[...] + b_ref[...], 0.0)
    wsig = jax.nn.sigmoid(
        jnp.dot(h, aw_ref[...], preferred_element_type=jnp.float32)
        + ab_ref[...])
    gid = gid_ref[0]
    grange = lax.broadcasted_iota(jnp.int32, (_G, _RBN), 0)
    mask = (grange == gid).astype(jnp.float32)
    upd = [jnp.dot(mask, h * wsig[:, t:t + 1],
                   precision=lax.Precision.HIGHEST,
                   preferred_element_type=jnp.float32) for t in range(_T)]
    o_ref[...] += jnp.stack(upd, axis=0)


def _final_readout(aggp, trans_prev, bias, awp, awb, gid3):
    return pl.pallas_call(
        _freadout_body,
        grid=(_RNB,),
        in_specs=[pl.BlockSpec((1, _RBN, _D), lambda i: (0, i, 0)),
                  pl.BlockSpec((1, _RBN, _D), lambda i: (1, i, 0)),
                  pl.BlockSpec((_RBN, _D), lambda i: (i, _R)),
                  pl.BlockSpec((1, _D), lambda i: (0, 0)),
                  pl.BlockSpec((_D, 128), lambda i: (0, 0)),
                  pl.BlockSpec((1, 128), lambda i: (0, 0)),
                  pl.BlockSpec((1, 1, _RBN), lambda i: (i, 0, 0))],
        out_specs=pl.BlockSpec((_T, _G, _D), lambda i: (0, 0, 0)),
        out_shape=jax.ShapeDtypeStruct((_T, _G, _D), jnp.float32),
    )(aggp, aggp, trans_prev, bias, awp, awb, gid3)


def _ffn_body(gf_ref, w1_ref, b1_ref, g1_ref, gb1_ref,
              w2_ref, b2_ref, g2_ref, gb2_ref,
              w3_ref, b3_ref, g3_ref, gb3_ref,
              wa_ref, ba_ref, wbp_ref, bbp_ref, o_ref):
    acc = jnp.zeros((_G, 128), jnp.float32)
    for t in range(_T):
        x = gf_ref[t]
        x = jnp.maximum(
            jnp.dot(x, w1_ref[t], preferred_element_type=jnp.float32)
            + b1_ref[t:t + 1], 0.0)
        x = x * g1_ref[t:t + 1] + gb1_ref[t:t + 1]
        x = jnp.maximum(
            jnp.dot(x, w2_ref[t], preferred_element_type=jnp.float32)
            + b2_ref[t:t + 1], 0.0)
        x = x * g2_ref[t:t + 1] + gb2_ref[t:t + 1]
        x = jnp.maximum(
            jnp.dot(x, w3_ref[t], preferred_element_type=jnp.float32)
            + b3_ref[t:t + 1], 0.0)
        x = x * g3_ref[t:t + 1] + gb3_ref[t:t + 1]
        hid = jnp.maximum(
            jnp.dot(x, wa_ref[t], preferred_element_type=jnp.float32)
            + ba_ref[t:t + 1], 0.0)
        acc = acc + jnp.dot(hid, wbp_ref[t],
                            preferred_element_type=jnp.float32)
    o_ref[...] = acc + bbp_ref[...]


def _ffn(gf, *ws):
    full = lambda a: pl.BlockSpec(a.shape, lambda: tuple(0 for _ in a.shape))
    return pl.pallas_call(
        _ffn_body,
        in_specs=[full(gf)] + [full(w) for w in ws],
        out_specs=pl.BlockSpec((_G, 128), lambda: (0, 0)),
        out_shape=jax.ShapeDtypeStruct((_G, 128), jnp.float32),
    )(gf, *ws)


def kernel(node_feats, edge_index, etype, graph_ids,
           Wrel1, Wself1, b1, Wrel2, Wself2, b2,
           aw_W, aw_b,
           fc1_W, fc1_b, bn1_g, bn1_b,
           fc2_W, fc2_b, bn2_g, bn2_b,
           fc3_W, fc3_b, bn3_g, bn3_b,
           outA_W, outA_b, outB_W, outB_b):
    f32 = jnp.float32
    gidx4 = _gather_idx(edge_index[0].reshape(_E // 128, 128),
                        etype.reshape(_E // 128, 128)
                        ).reshape(_NW, _P, _CPP, _C)
    dst4 = edge_index[1].reshape(_NW, _P, _CPP, _C)

    Wfull1 = jnp.concatenate(
        [Wrel1.transpose(1, 0, 2).reshape(_D, _R * _D), Wself1], axis=1)
    Wfull2 = jnp.concatenate(
        [Wrel2.transpose(1, 0, 2).reshape(_D, _R * _D), Wself2], axis=1)

    edge_agg = _make_edge_aggregate()

    trans1 = _dense_matmul(node_feats, Wfull1)
    aggp1 = edge_agg(trans1.reshape(_N * (_R + 1), _D), gidx4, dst4)
    trans2 = _layer_matmul(aggp1, trans1, b1.reshape(1, _D), Wfull2)
    aggp2 = edge_agg(trans2.reshape(_N * (_R + 1), _D), gidx4, dst4)

    awp = jnp.zeros((_D, 128), f32).at[:, :_T].set(aw_W.T)
    awb = jnp.zeros((1, 128), f32).at[0, :_T].set(aw_b)
    gid3 = graph_ids.reshape(_RNB, 1, _RBN)
    gf = _final_readout(aggp2, trans2, b2.reshape(1, _D),
                        awp, awb, gid3)

    s0 = (1.0 + 1e-5) ** -0.5
    wbp = outB_W[:, :, None] * jnp.eye(_T, 128, dtype=f32)[:, None, :]
    bbp = jnp.zeros((1, 128), f32).at[0, :_T].set(outB_b)
    pred = _ffn(gf, fc1_W, fc1_b, bn1_g * s0, bn1_b,
                fc2_W, fc2_b, bn2_g * s0, bn2_b,
                fc3_W, fc3_b, bn3_g * s0, bn3_b,
                outA_W, outA_b, wbp, bbp)
    return pred[:, :_T]


# speedup vs baseline: 32.1939x; 1.0156x over previous
import functools

import jax
import jax.numpy as jnp
from jax import lax
from jax.experimental import pallas as pl
from jax.experimental.pallas import tpu as pltpu
from jax.experimental.pallas import tpu_sc as plsc

_N = 10000
_E = 320000
_D = 128
_R = 4
_T = 3
_G = 256
_H = 128

_NC = 2
_NS = 16
_NW = _NC * _NS
_EPW = _E // _NW
_C = 40
_NCH = _EPW // _C
_P = 10
_CPP = _NCH // _P
_NBUF = 4
_NAGG = 10112
_RPS = _NAGG // _NS
_ZR = 8

@functools.cache
def _make_edge_aggregate():
    mesh = plsc.VectorSubcoreMesh(
        core_axis_name="c", subcore_axis_name="s",
        num_cores=_NC, num_subcores=_NS)
    return functools.partial(
        pl.kernel,
        out_type=jax.ShapeDtypeStruct((_NC, _NAGG, _D), jnp.float32),
        mesh=mesh,
        scratch_types=[
            pltpu.VMEM((_CPP, _C), jnp.int32),
            pltpu.VMEM((_CPP, _C), jnp.int32),
            pltpu.VMEM((_NBUF, _C, _D), jnp.float32),
            pltpu.VMEM((_ZR, _D), jnp.float32),
            pltpu.VMEM_SHARED((_NAGG, _D), jnp.float32),
            [pltpu.SemaphoreType.DMA] * _NBUF,
        ],
    )(_edge_aggregate_body)


def _edge_aggregate_body(tab, gidx_h, dst_h, out_h,
                         gib, dstb, rbuf, zbuf, agg, sems):
    c = lax.axis_index("c")
    s = lax.axis_index("s")
    wid = s * _NC + c

    zvec = jnp.zeros((16,), jnp.float32)

    def _zfill(i, carry):
        zbuf[i // 8, pl.ds((i % 8) * 16, 16)] = zvec
        return carry
    lax.fori_loop(0, _ZR * (_D // 16), _zfill, 0)

    def _zcp(j, carry):
        pltpu.sync_copy(zbuf, agg.at[pl.ds(s * _RPS + j * _ZR, _ZR), :])
        return carry
    lax.fori_loop(0, _RPS // _ZR, _zcp, 0)
    plsc.subcore_barrier()

    def _fire(j, b):
        pltpu.async_copy(tab.at[gib.at[j]], rbuf.at[b], sems[b])

    def _wait(b):
        pltpu.make_async_copy(tab.at[pl.ds(0, _C)], rbuf.at[b], sems[b]).wait()

    def _scat(j, b):
        pltpu.sync_copy(rbuf.at[b], agg.at[dstb.at[j]], add=True)

    def _pass(p, carry):
        pltpu.sync_copy(gidx_h.at[wid, p], gib)
        pltpu.sync_copy(dst_h.at[wid, p], dstb)

        for b in range(_NBUF):
            _fire(b, b)

        def _outer(o, carry2):
            for b in range(_NBUF):
                j = o * _NBUF + b
                _wait(b)
                _scat(j, b)
                jn = j + _NBUF

                @pl.when(jn < _CPP)
                def _():
                    _fire(jn, b)
            return carry2
        lax.fori_loop(0, _CPP // _NBUF, _outer, 0)
        for j in range((_CPP // _NBUF) * _NBUF, _CPP):
            _wait(j % _NBUF)
            _scat(j, j % _NBUF)
        return carry
    lax.fori_loop(0, _P, _pass, 0)

    plsc.subcore_barrier()
    pltpu.sync_copy(agg.at[pl.ds(s * _RPS, _RPS), :],
                    out_h.at[c, pl.ds(s * _RPS, _RPS), :])


_BN = 1000


def _mm_body(x_ref, w_ref, o_ref):
    o_ref[...] = jnp.dot(x_ref[...], w_ref[...],
                         preferred_element_type=jnp.float32)


def _dense_matmul(x, w):
    n, k = x.shape
    m = w.shape[1]
    return pl.pallas_call(
        _mm_body,
        grid=(n // _BN,),
        in_specs=[pl.BlockSpec((_BN, k), lambda i: (i, 0)),
                  pl.BlockSpec((k, m), lambda i: (0, 0))],
        out_specs=pl.BlockSpec((_BN, m), lambda i: (i, 0)),
        out_shape=jax.ShapeDtypeStruct((n, m), jnp.float32),
    )(x, w)


def _gidx_body(s_ref, e_ref, o_ref):
    o_ref[...] = s_ref[...] * (_R + 1) + e_ref[...]


def _gather_idx(src2, et2):
    full = pl.BlockSpec((_E // 128, 128), lambda: (0, 0))
    return pl.pallas_call(
        _gidx_body,
        in_specs=[full, full],
        out_specs=full,
        out_shape=jax.ShapeDtypeStruct((_E // 128, 128), jnp.int32),
    )(src2, et2)


def _layer_body(a0_ref, a1_ref, tp_ref, b_ref, w_ref, o_ref):
    h = jnp.maximum(a0_ref[0] + a1_ref[0] + tp_ref[...] + b_ref[...], 0.0)
    o_ref[...] = jnp.dot(h, w_ref[...], preferred_element_type=jnp.float32)


def _layer_matmul(aggp, trans_prev, bias, w):
    m = w.shape[1]
    return pl.pallas_call(
        _layer_body,
        grid=(_N // _BN,),
        in_specs=[pl.BlockSpec((1, _BN, _D), lambda i: (0, i, 0)),
                  pl.BlockSpec((1, _BN, _D), lambda i: (1, i, 0)),
                  pl.BlockSpec((_BN, _D), lambda i: (i, _R)),
                  pl.BlockSpec((1, _D), lambda i: (0, 0)),
                  pl.BlockSpec((_D, m), lambda i: (0, 0))],
        out_specs=pl.BlockSpec((_BN, m), lambda i: (i, 0)),
        out_shape=jax.ShapeDtypeStruct((_N, m), jnp.float32),
    )(aggp, aggp, trans_prev, bias, w)


_RBN = 1000
_RNB = _N // _RBN


def _freadout_body(a0_ref, a1_ref, tp_ref, b_ref, aw_ref, ab_ref, gid_ref,
                   o_ref):
    @pl.when(pl.program_id(0) == 0)
    def _():
        o_ref[...] = jnp.zeros_like(o_ref)

    h = jnp.maximum(a0_ref[0] + a1_ref[0] + tp_ref[...] + b_ref[...], 0.0)
    wsig = jax.nn.sigmoid(
        jnp.dot(h, aw_ref[...], preferred_element_type=jnp.float32)
        + ab_ref[...])
    gid = gid_ref[0]
    grange = lax.broadcasted_iota(jnp.int32, (_G, _RBN), 0)
    mask = (grange == gid).astype(jnp.float32)
    upd = [jnp.dot(mask, h * wsig[:, t:t + 1],
                   precision=lax.Precision.HIGHEST,
                   preferred_element_type=jnp.float32) for t in range(_T)]
    o_ref[...] += jnp.stack(upd, axis=0)


def _final_readout(aggp, trans_prev, bias, awp, awb, gid3):
    return pl.pallas_call(
        _freadout_body,
        grid=(_RNB,),
        in_specs=[pl.BlockSpec((1, _RBN, _D), lambda i: (0, i, 0)),
                  pl.BlockSpec((1, _RBN, _D), lambda i: (1, i, 0)),
                  pl.BlockSpec((_RBN, _D), lambda i: (i, _R)),
                  pl.BlockSpec((1, _D), lambda i: (0, 0)),
                  pl.BlockSpec((_D, 128), lambda i: (0, 0)),
                  pl.BlockSpec((1, 128), lambda i: (0, 0)),
                  pl.BlockSpec((1, 1, _RBN), lambda i: (i, 0, 0))],
        out_specs=pl.BlockSpec((_T, _G, _D), lambda i: (0, 0, 0)),
        out_shape=jax.ShapeDtypeStruct((_T, _G, _D), jnp.float32),
    )(aggp, aggp, trans_prev, bias, awp, awb, gid3)


def _ffn_body(gf_ref, w1_ref, b1_ref, g1_ref, gb1_ref,
              w2_ref, b2_ref, g2_ref, gb2_ref,
              w3_ref, b3_ref, g3_ref, gb3_ref,
              wa_ref, ba_ref, wbp_ref, bbp_ref, o_ref):
    acc = jnp.zeros((_G, 128), jnp.float32)
    for t in range(_T):
        x = gf_ref[t]
        x = jnp.maximum(
            jnp.dot(x, w1_ref[t], preferred_element_type=jnp.float32)
            + b1_ref[t:t + 1], 0.0)
        x = x * g1_ref[t:t + 1] + gb1_ref[t:t + 1]
        x = jnp.maximum(
            jnp.dot(x, w2_ref[t], preferred_element_type=jnp.float32)
            + b2_ref[t:t + 1], 0.0)
        x = x * g2_ref[t:t + 1] + gb2_ref[t:t + 1]
        x = jnp.maximum(
            jnp.dot(x, w3_ref[t], preferred_element_type=jnp.float32)
            + b3_ref[t:t + 1], 0.0)
        x = x * g3_ref[t:t + 1] + gb3_ref[t:t + 1]
        hid = jnp.maximum(
            jnp.dot(x, wa_ref[t], preferred_element_type=jnp.float32)
            + ba_ref[t:t + 1], 0.0)
        acc = acc + jnp.dot(hid, wbp_ref[t],
                            preferred_element_type=jnp.float32)
    o_ref[...] = acc + bbp_ref[...]


def _ffn(gf, *ws):
    full = lambda a: pl.BlockSpec(a.shape, lambda: tuple(0 for _ in a.shape))
    return pl.pallas_call(
        _ffn_body,
        in_specs=[full(gf)] + [full(w) for w in ws],
        out_specs=pl.BlockSpec((_G, 128), lambda: (0, 0)),
        out_shape=jax.ShapeDtypeStruct((_G, 128), jnp.float32),
    )(gf, *ws)


def kernel(node_feats, edge_index, etype, graph_ids,
           Wrel1, Wself1, b1, Wrel2, Wself2, b2,
           aw_W, aw_b,
           fc1_W, fc1_b, bn1_g, bn1_b,
           fc2_W, fc2_b, bn2_g, bn2_b,
           fc3_W, fc3_b, bn3_g, bn3_b,
           outA_W, outA_b, outB_W, outB_b):
    f32 = jnp.float32
    gidx4 = _gather_idx(edge_index[0].reshape(_E // 128, 128),
                        etype.reshape(_E // 128, 128)
                        ).reshape(_NW, _P, _CPP, _C)
    dst4 = edge_index[1].reshape(_NW, _P, _CPP, _C)

    Wfull1 = jnp.concatenate(
        [Wrel1.transpose(1, 0, 2).reshape(_D, _R * _D), Wself1], axis=1)
    Wfull2 = jnp.concatenate(
        [Wrel2.transpose(1, 0, 2).reshape(_D, _R * _D), Wself2], axis=1)

    edge_agg = _make_edge_aggregate()

    trans1 = _dense_matmul(node_feats, Wfull1)
    aggp1 = edge_agg(trans1.reshape(_N * (_R + 1), _D), gidx4, dst4)
    trans2 = _layer_matmul(aggp1, trans1, b1.reshape(1, _D), Wfull2)
    aggp2 = edge_agg(trans2.reshape(_N * (_R + 1), _D), gidx4, dst4)

    awp = jnp.zeros((_D, 128), f32).at[:, :_T].set(aw_W.T)
    awb = jnp.zeros((1, 128), f32).at[0, :_T].set(aw_b)
    gid3 = graph_ids.reshape(_RNB, 1, _RBN)
    gf = _final_readout(aggp2, trans2, b2.reshape(1, _D),
                        awp, awb, gid3)

    s0 = (1.0 + 1e-5) ** -0.5
    wbp = outB_W[:, :, None] * jnp.eye(_T, 128, dtype=f32)[:, None, :]
    bbp = jnp.zeros((1, 128), f32).at[0, :_T].set(outB_b)
    pred = _ffn(gf, fc1_W, fc1_b, bn1_g * s0, bn1_b,
                fc2_W, fc2_b, bn2_g * s0, bn2_b,
                fc3_W, fc3_b, bn3_g * s0, bn3_b,
                outA_W, outA_b, wbp, bbp)
    return pred[:, :_T]


# speedup vs baseline: 35.1385x; 1.0915x over previous
import functools

import jax
import jax.numpy as jnp
from jax import lax
from jax.experimental import pallas as pl
from jax.experimental.pallas import tpu as pltpu
from jax.experimental.pallas import tpu_sc as plsc

_N = 10000
_E = 320000
_D = 128
_R = 4
_T = 3
_G = 256
_H = 128

_NC = 2
_NS = 16
_NW = _NC * _NS
_EPW = _E // _NW
_C = 40
_NCH = _EPW // _C
_P = 5
_CPP = _NCH // _P
_NBUF = 5
_NAGG = 10112
_RPS = _NAGG // _NS
_ZR = 8

@functools.cache
def _make_edge_aggregate():
    mesh = plsc.VectorSubcoreMesh(
        core_axis_name="c", subcore_axis_name="s",
        num_cores=_NC, num_subcores=_NS)
    return functools.partial(
        pl.kernel,
        out_type=jax.ShapeDtypeStruct((_NC, _NAGG, _D), jnp.float32),
        mesh=mesh,
        scratch_types=[
            pltpu.VMEM((_CPP, _C), jnp.int32),
            pltpu.VMEM((_CPP, _C), jnp.int32),
            pltpu.VMEM((_NBUF, _C, _D), jnp.float32),
            pltpu.VMEM((_ZR, _D), jnp.float32),
            pltpu.VMEM_SHARED((_NAGG, _D), jnp.float32),
            [pltpu.SemaphoreType.DMA] * _NBUF,
        ],
    )(_edge_aggregate_body)


def _edge_aggregate_body(tab, gidx_h, dst_h, out_h,
                         gib, dstb, rbuf, zbuf, agg, sems):
    c = lax.axis_index("c")
    s = lax.axis_index("s")
    wid = s * _NC + c

    zvec = jnp.zeros((16,), jnp.float32)

    def _zfill(i, carry):
        zbuf[i // 8, pl.ds((i % 8) * 16, 16)] = zvec
        return carry
    lax.fori_loop(0, _ZR * (_D // 16), _zfill, 0)

    def _zcp(j, carry):
        pltpu.sync_copy(zbuf, agg.at[pl.ds(s * _RPS + j * _ZR, _ZR), :])
        return carry
    lax.fori_loop(0, _RPS // _ZR, _zcp, 0)
    plsc.subcore_barrier()

    def _fire(j, b):
        pltpu.async_copy(tab.at[gib.at[j]], rbuf.at[b], sems[b])

    def _wait(b):
        pltpu.make_async_copy(tab.at[pl.ds(0, _C)], rbuf.at[b], sems[b]).wait()

    def _scat(j, b):
        pltpu.sync_copy(rbuf.at[b], agg.at[dstb.at[j]], add=True)

    def _pass(p, carry):
        pltpu.sync_copy(gidx_h.at[wid, p], gib)
        pltpu.sync_copy(dst_h.at[wid, p], dstb)

        for b in range(_NBUF):
            _fire(b, b)

        def _outer(o, carry2):
            for b in range(_NBUF):
                j = o * _NBUF + b
                _wait(b)
                _scat(j, b)
                jn = j + _NBUF

                @pl.when(jn < _CPP)
                def _():
                    _fire(jn, b)
            return carry2
        lax.fori_loop(0, _CPP // _NBUF, _outer, 0)
        for j in range((_CPP // _NBUF) * _NBUF, _CPP):
            _wait(j % _NBUF)
            _scat(j, j % _NBUF)
        return carry
    lax.fori_loop(0, _P, _pass, 0)

    plsc.subcore_barrier()
    pltpu.sync_copy(agg.at[pl.ds(s * _RPS, _RPS), :],
                    out_h.at[c, pl.ds(s * _RPS, _RPS), :])


_BN = 1000


def _mm_body(x_ref, w_ref, o_ref):
    o_ref[...] = jnp.dot(x_ref[...], w_ref[...],
                         preferred_element_type=jnp.float32)


def _dense_matmul(x, w):
    n, k = x.shape
    m = w.shape[1]
    return pl.pallas_call(
        _mm_body,
        grid=(n // _BN,),
        in_specs=[pl.BlockSpec((_BN, k), lambda i: (i, 0)),
                  pl.BlockSpec((k, m), lambda i: (0, 0))],
        out_specs=pl.BlockSpec((_BN, m), lambda i: (i, 0)),
        out_shape=jax.ShapeDtypeStruct((n, m), jnp.float32),
    )(x, w)


def _gidx_body(s_ref, e_ref, o_ref):
    o_ref[...] = s_ref[...] * (_R + 1) + e_ref[...]


def _gather_idx(src2, et2):
    full = pl.BlockSpec((_E // 128, 128), lambda: (0, 0))
    return pl.pallas_call(
        _gidx_body,
        in_specs=[full, full],
        out_specs=full,
        out_shape=jax.ShapeDtypeStruct((_E // 128, 128), jnp.int32),
    )(src2, et2)


def _layer_body(a0_ref, a1_ref, tp_ref, b_ref, w_ref, o_ref):
    h = jnp.maximum(a0_ref[0] + a1_ref[0] + tp_ref[...] + b_ref[...], 0.0)
    o_ref[...] = jnp.dot(h, w_ref[...], preferred_element_type=jnp.float32)


def _layer_matmul(aggp, trans_prev, bias, w):
    m = w.shape[1]
    return pl.pallas_call(
        _layer_body,
        grid=(_N // _BN,),
        in_specs=[pl.BlockSpec((1, _BN, _D), lambda i: (0, i, 0)),
                  pl.BlockSpec((1, _BN, _D), lambda i: (1, i, 0)),
                  pl.BlockSpec((_BN, _D), lambda i: (i, _R)),
                  pl.BlockSpec((1, _D), lambda i: (0, 0)),
                  pl.BlockSpec((_D, m), lambda i: (0, 0))],
        out_specs=pl.BlockSpec((_BN, m), lambda i: (i, 0)),
        out_shape=jax.ShapeDtypeStruct((_N, m), jnp.float32),
    )(aggp, aggp, trans_prev, bias, w)


_RBN = 1000
_RNB = _N // _RBN


def _freadout_body(a0_ref, a1_ref, tp_ref, b_ref, aw_ref, ab_ref, gid_ref,
                   o_ref):
    @pl.when(pl.program_id(0) == 0)
    def _():
        o_ref[...] = jnp.zeros_like(o_ref)

    h = jnp.maximum(a0_ref[0] + a1_ref[0] + tp_ref[...] + b_ref[...], 0.0)
    wsig = jax.nn.sigmoid(
        jnp.dot(h, aw_ref[...], preferred_element_type=jnp.float32)
        + ab_ref[...])
    gid = gid_ref[0]
    grange = lax.broadcasted_iota(jnp.int32, (_G, _RBN), 0)
    mask = (grange == gid).astype(jnp.float32)
    upd = [jnp.dot(mask, h * wsig[:, t:t + 1],
                   precision=lax.Precision.HIGHEST,
                   preferred_element_type=jnp.float32) for t in range(_T)]
    o_ref[...] += jnp.stack(upd, axis=0)


def _final_readout(aggp, trans_prev, bias, awp, awb, gid3):
    return pl.pallas_call(
        _freadout_body,
        grid=(_RNB,),
        in_specs=[pl.BlockSpec((1, _RBN, _D), lambda i: (0, i, 0)),
                  pl.BlockSpec((1, _RBN, _D), lambda i: (1, i, 0)),
                  pl.BlockSpec((_RBN, _D), lambda i: (i, _R)),
                  pl.BlockSpec((1, _D), lambda i: (0, 0)),
                  pl.BlockSpec((_D, 128), lambda i: (0, 0)),
                  pl.BlockSpec((1, 128), lambda i: (0, 0)),
                  pl.BlockSpec((1, 1, _RBN), lambda i: (i, 0, 0))],
        out_specs=pl.BlockSpec((_T, _G, _D), lambda i: (0, 0, 0)),
        out_shape=jax.ShapeDtypeStruct((_T, _G, _D), jnp.float32),
    )(aggp, aggp, trans_prev, bias, awp, awb, gid3)


def _ffn_body(gf_ref, w1_ref, b1_ref, g1_ref, gb1_ref,
              w2_ref, b2_ref, g2_ref, gb2_ref,
              w3_ref, b3_ref, g3_ref, gb3_ref,
              wa_ref, ba_ref, wbp_ref, bbp_ref, o_ref):
    acc = jnp.zeros((_G, 128), jnp.float32)
    for t in range(_T):
        x = gf_ref[t]
        x = jnp.maximum(
            jnp.dot(x, w1_ref[t], preferred_element_type=jnp.float32)
            + b1_ref[t:t + 1], 0.0)
        x = x * g1_ref[t:t + 1] + gb1_ref[t:t + 1]
        x = jnp.maximum(
            jnp.dot(x, w2_ref[t], preferred_element_type=jnp.float32)
            + b2_ref[t:t + 1], 0.0)
        x = x * g2_ref[t:t + 1] + gb2_ref[t:t + 1]
        x = jnp.maximum(
            jnp.dot(x, w3_ref[t], preferred_element_type=jnp.float32)
            + b3_ref[t:t + 1], 0.0)
        x = x * g3_ref[t:t + 1] + gb3_ref[t:t + 1]
        hid = jnp.maximum(
            jnp.dot(x, wa_ref[t], preferred_element_type=jnp.float32)
            + ba_ref[t:t + 1], 0.0)
        acc = acc + jnp.dot(hid, wbp_ref[t],
                            preferred_element_type=jnp.float32)
    o_ref[...] = acc + bbp_ref[...]


def _ffn(gf, *ws):
    full = lambda a: pl.BlockSpec(a.shape, lambda: tuple(0 for _ in a.shape))
    return pl.pallas_call(
        _ffn_body,
        in_specs=[full(gf)] + [full(w) for w in ws],
        out_specs=pl.BlockSpec((_G, 128), lambda: (0, 0)),
        out_shape=jax.ShapeDtypeStruct((_G, 128), jnp.float32),
    )(gf, *ws)


def kernel(node_feats, edge_index, etype, graph_ids,
           Wrel1, Wself1, b1, Wrel2, Wself2, b2,
           aw_W, aw_b,
           fc1_W, fc1_b, bn1_g, bn1_b,
           fc2_W, fc2_b, bn2_g, bn2_b,
           fc3_W, fc3_b, bn3_g, bn3_b,
           outA_W, outA_b, outB_W, outB_b):
    f32 = jnp.float32
    gidx4 = _gather_idx(edge_index[0].reshape(_E // 128, 128),
                        etype.reshape(_E // 128, 128)
                        ).reshape(_NW, _P, _CPP, _C)
    dst4 = edge_index[1].reshape(_NW, _P, _CPP, _C)

    Wfull1 = jnp.concatenate(
        [Wrel1.transpose(1, 0, 2).reshape(_D, _R * _D), Wself1], axis=1)
    Wfull2 = jnp.concatenate(
        [Wrel2.transpose(1, 0, 2).reshape(_D, _R * _D), Wself2], axis=1)

    edge_agg = _make_edge_aggregate()

    trans1 = _dense_matmul(node_feats, Wfull1)
    aggp1 = edge_agg(trans1.reshape(_N * (_R + 1), _D), gidx4, dst4)
    trans2 = _layer_matmul(aggp1, trans1, b1.reshape(1, _D), Wfull2)
    aggp2 = edge_agg(trans2.reshape(_N * (_R + 1), _D), gidx4, dst4)

    awp = jnp.zeros((_D, 128), f32).at[:, :_T].set(aw_W.T)
    awb = jnp.zeros((1, 128), f32).at[0, :_T].set(aw_b)
    gid3 = graph_ids.reshape(_RNB, 1, _RBN)
    gf = _final_readout(aggp2, trans2, b2.reshape(1, _D),
                        awp, awb, gid3)

    s0 = (1.0 + 1e-5) ** -0.5
    wbp = outB_W[:, :, None] * jnp.eye(_T, 128, dtype=f32)[:, None, :]
    bbp = jnp.zeros((1, 128), f32).at[0, :_T].set(outB_b)
    pred = _ffn(gf, fc1_W, fc1_b, bn1_g * s0, bn1_b,
                fc2_W, fc2_b, bn2_g * s0, bn2_b,
                fc3_W, fc3_b, bn3_g * s0, bn3_b,
                outA_W, outA_b, wbp, bbp)
    return pred[:, :_T]
